# Initial kernel scaffold; baseline (speedup 1.0000x reference)
import jax
import jax.numpy as jnp
from jax.experimental import pallas as pl


def kernel(inputs, params):
    raise NotImplementedError("write your pallas kernel here")


# speedup vs baseline: 2.4670x; 2.4670x over previous
import jax
import jax.numpy as jnp
from jax.experimental import pallas as pl
from jax.experimental.pallas import tpu as pltpu

_EPS = 1e-6
_K = 20
_B, _N = 8, 2048
_TQ = 512


def _knn_edge_kernel(pts_ref, xq_ref, m_ref, out_ref):
    pts = pts_ref[0]
    xq = xq_ref[0]
    mw = m_ref[...]

    dotq = jax.lax.dot_general(xq, pts, (((1,), (1,)), ((), ())),
                               preferred_element_type=jnp.float32)
    inner = -2.0 * dotq
    xxq = jnp.sum(xq * xq, axis=1, keepdims=True)
    sq = pts * pts
    ones13 = jnp.ones((1, 3), jnp.float32)
    xxall = jax.lax.dot_general(ones13, sq, (((1,), (1,)), ((), ())),
                                preferred_element_type=jnp.float32,
                                precision=jax.lax.Precision.HIGHEST)
    pd = -xxq - inner - xxall

    iota = jax.lax.broadcasted_iota(jnp.int32, pd.shape, 1)
    big = jnp.int32(_N)
    neg_inf = jnp.float32(-jnp.inf)

    xqx = xq[:, 0:1]
    xqy = xq[:, 1:2]
    xqz = xq[:, 2:3]

    work = pd
    acc0 = jnp.zeros((_TQ, 64), jnp.float32)
    acc1 = jnp.zeros((_TQ, 64), jnp.float32)
    acc2 = jnp.zeros((_TQ, 64), jnp.float32)
    for _ in range(_K):
        mx = jnp.max(work, axis=1, keepdims=True)
        cand = jnp.where(work == mx, iota, big)
        am = jnp.min(cand, axis=1, keepdims=True)
        hit = cand == am
        work = jnp.where(hit, neg_inf, work)
        onehot = hit.astype(jnp.float32)
        f = jnp.dot(onehot, pts, preferred_element_type=jnp.float32,
                    precision=jax.lax.Precision.HIGHEST)
        fx = f[:, 0:1]
        fy = f[:, 1:2]
        fz = f[:, 2:3]
        dx = fx - xqx
        dy = fy - xqy
        dz = fz - xqz
        cx = fy * xqz - fz * xqy
        cy = fz * xqx - fx * xqz
        cz = fx * xqy - fy * xqx
        g = jnp.concatenate([dx, dy, dz, xqx, xqy, xqz, cx, cy, cz],
                            axis=1)
        pdm = jnp.dot(g, mw, preferred_element_type=jnp.float32)
        p0 = pdm[:, 0:64]
        d0 = pdm[:, 64:128]
        p1 = pdm[:, 128:192]
        d1 = pdm[:, 192:256]
        p2 = pdm[:, 256:320]
        d2v = pdm[:, 320:384]
        dot = p0 * d0 + p1 * d1 + p2 * d2v
        dd = d0 * d0 + d1 * d1 + d2v * d2v
        coef = dot / (dd + _EPS)
        keep = dot >= 0
        acc0 = acc0 + jnp.where(keep, p0, p0 - coef * d0)
        acc1 = acc1 + jnp.where(keep, p1, p1 - coef * d1)
        acc2 = acc2 + jnp.where(keep, p2, p2 - coef * d2v)

    scale = jnp.float32(1.0 / _K)
    out_ref[0] = jnp.concatenate([acc0, acc1, acc2], axis=1) * scale


def _vnlr(x, d_t, n):
    d = jnp.dot(x, d_t, preferred_element_type=jnp.float32)
    z = x * d
    dot = z[0:n] + z[n:2 * n] + z[2 * n:3 * n]
    zz = d * d
    d2 = zz[0:n] + zz[n:2 * n] + zz[2 * n:3 * n]
    coef = dot / (d2 + _EPS)
    dot3 = jnp.concatenate([dot, dot, dot], axis=0)
    coef3 = jnp.concatenate([coef, coef, coef], axis=0)
    return jnp.where(dot3 >= 0, x, x - coef3 * d)


def _resblock(x, a0t, f0t, a1t, f1t, sct, n):
    net = jnp.dot(_vnlr(x, a0t, n), f0t, preferred_element_type=jnp.float32)
    dx = jnp.dot(_vnlr(net, a1t, n), f1t, preferred_element_type=jnp.float32)
    return jnp.dot(x, sct, preferred_element_type=jnp.float32) + dx


def _poolcat(x, n):
    m0 = jnp.mean(x[0:n], axis=0, keepdims=True)
    m1 = jnp.mean(x[n:2 * n], axis=0, keepdims=True)
    m2 = jnp.mean(x[2 * n:3 * n], axis=0, keepdims=True)
    c = x.shape[1]
    pooled = jnp.concatenate([
        jnp.broadcast_to(m0, (n, c)),
        jnp.broadcast_to(m1, (n, c)),
        jnp.broadcast_to(m2, (n, c)),
    ], axis=0)
    return jnp.concatenate([x, pooled], axis=1)


def _dense_kernel(net0_ref, fcpos_t_ref, *rest):
    wrefs = rest[:-3]
    actc_t_ref, fcc_t_ref, out_ref = rest[-3:]
    n = _N
    x = net0_ref[0]
    x = jnp.concatenate([x[:, 0:64], x[:, 64:128], x[:, 128:192]],
                        axis=0)
    net = jnp.dot(x, fcpos_t_ref[...], preferred_element_type=jnp.float32)
    for i in range(5):
        a0t = wrefs[5 * i][...]
        f0t = wrefs[5 * i + 1][...]
        a1t = wrefs[5 * i + 2][...]
        f1t = wrefs[5 * i + 3][...]
        sct = wrefs[5 * i + 4][...]
        net = _resblock(net, a0t, f0t, a1t, f1t, sct, n)
        if i < 4:
            net = _poolcat(net, n)
    m0 = jnp.mean(net[0:n], axis=0, keepdims=True)
    m1 = jnp.mean(net[n:2 * n], axis=0, keepdims=True)
    m2 = jnp.mean(net[2 * n:3 * n], axis=0, keepdims=True)
    m3 = jnp.concatenate([m0, m1, m2], axis=0)
    d = jnp.dot(m3, actc_t_ref[...], preferred_element_type=jnp.float32)
    z = m3 * d
    dot = jnp.sum(z, axis=0, keepdims=True)
    d2 = jnp.sum(d * d, axis=0, keepdims=True)
    out = jnp.where(dot >= 0, m3, m3 - (dot / (d2 + _EPS)) * d)
    out_ref[0] = jnp.dot(out, fcc_t_ref[...],
                         preferred_element_type=jnp.float32)


def kernel(inputs, params):
    pts = inputs

    wf = params['conv_pos_feat']
    wd = params['conv_pos_dir']
    w2 = jnp.stack([wf.T, wd.T], axis=1)
    eye3 = jnp.eye(3, dtype=jnp.float32)
    mw = jnp.einsum('cso,vw->cvwso', w2, eye3).reshape(9, 384)

    nt = _N // _TQ
    net0 = pl.pallas_call(
        _knn_edge_kernel,
        grid=(_B, nt),
        in_specs=[
            pl.BlockSpec((1, _N, 3), lambda b, t: (b, 0, 0)),
            pl.BlockSpec((1, _TQ, 3), lambda b, t: (b, t, 0)),
            pl.BlockSpec((9, 384), lambda b, t: (0, 0)),
        ],
        out_specs=pl.BlockSpec((1, _TQ, 192), lambda b, t: (b, t, 0)),
        out_shape=jax.ShapeDtypeStruct((_B, _N, 192), jnp.float32),
    )(pts, pts, mw)

    wlist = [params['fc_pos'].T]
    for blk in params['blocks']:
        wlist += [blk['act0'].T, blk['fc0'].T, blk['act1'].T,
                  blk['fc1'].T, blk['sc'].T]
    wlist += [params['actc_dir'].T, params['fc_c'].T]

    wspecs = [pl.BlockSpec(w.shape, lambda b: (0,) * w.ndim) for w in wlist]
    cvec = pl.pallas_call(
        _dense_kernel,
        grid=(_B,),
        in_specs=[pl.BlockSpec((1, _N, 192), lambda b: (b, 0, 0))] + wspecs,
        out_specs=pl.BlockSpec((1, 3, 128), lambda b: (b, 0, 0)),
        out_shape=jax.ShapeDtypeStruct((_B, 3, 128), jnp.float32),
    )(net0, *wlist)

    return jnp.transpose(cvec, (0, 2, 1)).reshape(_B, 384)


# speedup vs baseline: 5.0881x; 2.0625x over previous
import jax
import jax.numpy as jnp
from jax.experimental import pallas as pl
from jax.experimental.pallas import tpu as pltpu

_EPS = 1e-6
_K = 20
_B, _N = 8, 2048
_TQ = 512


def _knn_edge_kernel(pts_ref, xq_ref, m_ref, out_ref):
    pts = pts_ref[0]
    xq = xq_ref[0]
    mw = m_ref[...]

    dotq = jax.lax.dot_general(xq, pts, (((1,), (1,)), ((), ())),
                               preferred_element_type=jnp.float32)
    inner = -2.0 * dotq
    xxq = jnp.sum(xq * xq, axis=1, keepdims=True)
    sq = pts * pts
    ones13 = jnp.ones((1, 3), jnp.float32)
    xxall = jax.lax.dot_general(ones13, sq, (((1,), (1,)), ((), ())),
                                preferred_element_type=jnp.float32,
                                precision=jax.lax.Precision.HIGHEST)
    pd = -xxq - inner - xxall

    pts_hi = pts.astype(jnp.bfloat16).astype(jnp.float32)
    pts_lo = pts - pts_hi

    iota = jax.lax.broadcasted_iota(jnp.int32, pd.shape, 1)
    big = jnp.int32(_N)
    neg_inf = jnp.float32(-jnp.inf)

    xqx = xq[:, 0:1]
    xqy = xq[:, 1:2]
    xqz = xq[:, 2:3]

    work = pd
    acc0 = jnp.zeros((_TQ, 64), jnp.float32)
    acc1 = jnp.zeros((_TQ, 64), jnp.float32)
    acc2 = jnp.zeros((_TQ, 64), jnp.float32)
    for _ in range(_K):
        mx = jnp.max(work, axis=1, keepdims=True)
        cand = jnp.where(work == mx, iota, big)
        am = jnp.min(cand, axis=1, keepdims=True)
        hit = cand == am
        work = jnp.where(hit, neg_inf, work)
        onehot = hit.astype(jnp.float32)
        f = (jnp.dot(onehot, pts_hi, preferred_element_type=jnp.float32)
             + jnp.dot(onehot, pts_lo,
                       preferred_element_type=jnp.float32))
        fx = f[:, 0:1]
        fy = f[:, 1:2]
        fz = f[:, 2:3]
        dx = fx - xqx
        dy = fy - xqy
        dz = fz - xqz
        cx = fy * xqz - fz * xqy
        cy = fz * xqx - fx * xqz
        cz = fx * xqy - fy * xqx
        g = jnp.concatenate([dx, dy, dz, xqx, xqy, xqz, cx, cy, cz],
                            axis=1)
        pdm = jnp.dot(g, mw, preferred_element_type=jnp.float32)
        p0 = pdm[:, 0:64]
        d0 = pdm[:, 64:128]
        p1 = pdm[:, 128:192]
        d1 = pdm[:, 192:256]
        p2 = pdm[:, 256:320]
        d2v = pdm[:, 320:384]
        dot = p0 * d0 + p1 * d1 + p2 * d2v
        dd = d0 * d0 + d1 * d1 + d2v * d2v
        coef = dot / (dd + _EPS)
        keep = dot >= 0
        acc0 = acc0 + jnp.where(keep, p0, p0 - coef * d0)
        acc1 = acc1 + jnp.where(keep, p1, p1 - coef * d1)
        acc2 = acc2 + jnp.where(keep, p2, p2 - coef * d2v)

    scale = jnp.float32(1.0 / _K)
    out_ref[0] = jnp.concatenate([acc0, acc1, acc2], axis=1) * scale


def _vnlr(x, d_t, n):
    d = jnp.dot(x, d_t, preferred_element_type=jnp.float32)
    z = x * d
    dot = z[0:n] + z[n:2 * n] + z[2 * n:3 * n]
    zz = d * d
    d2 = zz[0:n] + zz[n:2 * n] + zz[2 * n:3 * n]
    coef = dot / (d2 + _EPS)
    dot3 = jnp.concatenate([dot, dot, dot], axis=0)
    coef3 = jnp.concatenate([coef, coef, coef], axis=0)
    return jnp.where(dot3 >= 0, x, x - coef3 * d)


def _resblock(x, a0t, f0t, a1t, f1t, sct, n):
    net = jnp.dot(_vnlr(x, a0t, n), f0t, preferred_element_type=jnp.float32)
    dx = jnp.dot(_vnlr(net, a1t, n), f1t, preferred_element_type=jnp.float32)
    return jnp.dot(x, sct, preferred_element_type=jnp.float32) + dx


def _poolcat(x, n):
    m0 = jnp.mean(x[0:n], axis=0, keepdims=True)
    m1 = jnp.mean(x[n:2 * n], axis=0, keepdims=True)
    m2 = jnp.mean(x[2 * n:3 * n], axis=0, keepdims=True)
    c = x.shape[1]
    pooled = jnp.concatenate([
        jnp.broadcast_to(m0, (n, c)),
        jnp.broadcast_to(m1, (n, c)),
        jnp.broadcast_to(m2, (n, c)),
    ], axis=0)
    return jnp.concatenate([x, pooled], axis=1)


def _dense_kernel(net0_ref, fcpos_t_ref, *rest):
    wrefs = rest[:-3]
    actc_t_ref, fcc_t_ref, out_ref = rest[-3:]
    n = _N
    x = net0_ref[0]
    x = jnp.concatenate([x[:, 0:64], x[:, 64:128], x[:, 128:192]],
                        axis=0)
    net = jnp.dot(x, fcpos_t_ref[...], preferred_element_type=jnp.float32)
    for i in range(5):
        a0t = wrefs[5 * i][...]
        f0t = wrefs[5 * i + 1][...]
        a1t = wrefs[5 * i + 2][...]
        f1t = wrefs[5 * i + 3][...]
        sct = wrefs[5 * i + 4][...]
        net = _resblock(net, a0t, f0t, a1t, f1t, sct, n)
        if i < 4:
            net = _poolcat(net, n)
    m0 = jnp.mean(net[0:n], axis=0, keepdims=True)
    m1 = jnp.mean(net[n:2 * n], axis=0, keepdims=True)
    m2 = jnp.mean(net[2 * n:3 * n], axis=0, keepdims=True)
    m3 = jnp.concatenate([m0, m1, m2], axis=0)
    d = jnp.dot(m3, actc_t_ref[...], preferred_element_type=jnp.float32)
    z = m3 * d
    dot = jnp.sum(z, axis=0, keepdims=True)
    d2 = jnp.sum(d * d, axis=0, keepdims=True)
    out = jnp.where(dot >= 0, m3, m3 - (dot / (d2 + _EPS)) * d)
    out_ref[0] = jnp.dot(out, fcc_t_ref[...],
                         preferred_element_type=jnp.float32)


def kernel(inputs, params):
    pts = inputs

    wf = params['conv_pos_feat']
    wd = params['conv_pos_dir']
    w2 = jnp.stack([wf.T, wd.T], axis=1)
    eye3 = jnp.eye(3, dtype=jnp.float32)
    mw = jnp.einsum('cso,vw->cvwso', w2, eye3).reshape(9, 384)

    nt = _N // _TQ
    net0 = pl.pallas_call(
        _knn_edge_kernel,
        grid=(_B, nt),
        in_specs=[
            pl.BlockSpec((1, _N, 3), lambda b, t: (b, 0, 0)),
            pl.BlockSpec((1, _TQ, 3), lambda b, t: (b, t, 0)),
            pl.BlockSpec((9, 384), lambda b, t: (0, 0)),
        ],
        out_specs=pl.BlockSpec((1, _TQ, 192), lambda b, t: (b, t, 0)),
        out_shape=jax.ShapeDtypeStruct((_B, _N, 192), jnp.float32),
    )(pts, pts, mw)

    wlist = [params['fc_pos'].T]
    for blk in params['blocks']:
        wlist += [blk['act0'].T, blk['fc0'].T, blk['act1'].T,
                  blk['fc1'].T, blk['sc'].T]
    wlist += [params['actc_dir'].T, params['fc_c'].T]

    wspecs = [pl.BlockSpec(w.shape, lambda b: (0,) * w.ndim) for w in wlist]
    cvec = pl.pallas_call(
        _dense_kernel,
        grid=(_B,),
        in_specs=[pl.BlockSpec((1, _N, 192), lambda b: (b, 0, 0))] + wspecs,
        out_specs=pl.BlockSpec((1, 3, 128), lambda b: (b, 0, 0)),
        out_shape=jax.ShapeDtypeStruct((_B, 3, 128), jnp.float32),
    )(net0, *wlist)

    return jnp.transpose(cvec, (0, 2, 1)).reshape(_B, 384)


# speedup vs baseline: 6.4682x; 1.2712x over previous
import jax
import jax.numpy as jnp
from jax.experimental import pallas as pl
from jax.experimental.pallas import tpu as pltpu

_EPS = 1e-6
_K = 20
_B, _N = 8, 2048
_TQ = 512


def _knn_edge_kernel(pts_ref, xq_ref, m_ref, out_ref):
    pts = pts_ref[0]
    xq = xq_ref[0]
    mw = m_ref[...]

    dotq = jax.lax.dot_general(xq, pts, (((1,), (1,)), ((), ())),
                               preferred_element_type=jnp.float32)
    inner = -2.0 * dotq
    xxq = jnp.sum(xq * xq, axis=1, keepdims=True)
    sq = pts * pts
    ones13 = jnp.ones((1, 3), jnp.float32)
    xxall = jax.lax.dot_general(ones13, sq, (((1,), (1,)), ((), ())),
                                preferred_element_type=jnp.float32,
                                precision=jax.lax.Precision.HIGHEST)
    pd = -xxq - inner - xxall

    pts_hi = pts.astype(jnp.bfloat16).astype(jnp.float32)
    pts_hilo = jnp.concatenate([pts_hi, pts - pts_hi], axis=1)

    iota = jax.lax.broadcasted_iota(jnp.int32, pd.shape, 1)
    big = jnp.int32(_N)
    neg_inf = jnp.float32(-jnp.inf)

    xqx = xq[:, 0:1]
    xqy = xq[:, 1:2]
    xqz = xq[:, 2:3]

    work = pd
    acc0 = jnp.zeros((_TQ, 64), jnp.float32)
    acc1 = jnp.zeros((_TQ, 64), jnp.float32)
    acc2 = jnp.zeros((_TQ, 64), jnp.float32)
    for _ in range(_K):
        mx = jnp.max(work, axis=1, keepdims=True)
        cand = jnp.where(work == mx, iota, big)
        am = jnp.min(cand, axis=1, keepdims=True)
        hit = cand == am
        work = jnp.where(hit, neg_inf, work)
        onehot = hit.astype(jnp.float32)
        f6 = jnp.dot(onehot, pts_hilo,
                     preferred_element_type=jnp.float32)
        f = f6[:, 0:3] + f6[:, 3:6]
        fx = f[:, 0:1]
        fy = f[:, 1:2]
        fz = f[:, 2:3]
        dx = fx - xqx
        dy = fy - xqy
        dz = fz - xqz
        cx = fy * xqz - fz * xqy
        cy = fz * xqx - fx * xqz
        cz = fx * xqy - fy * xqx
        g = jnp.concatenate([dx, dy, dz, xqx, xqy, xqz, cx, cy, cz],
                            axis=1)
        pdm = jnp.dot(g, mw, preferred_element_type=jnp.float32)
        p0 = pdm[:, 0:64]
        d0 = pdm[:, 64:128]
        p1 = pdm[:, 128:192]
        d1 = pdm[:, 192:256]
        p2 = pdm[:, 256:320]
        d2v = pdm[:, 320:384]
        dot = p0 * d0 + p1 * d1 + p2 * d2v
        dd = d0 * d0 + d1 * d1 + d2v * d2v
        coef = dot / (dd + _EPS)
        keep = dot >= 0
        acc0 = acc0 + jnp.where(keep, p0, p0 - coef * d0)
        acc1 = acc1 + jnp.where(keep, p1, p1 - coef * d1)
        acc2 = acc2 + jnp.where(keep, p2, p2 - coef * d2v)

    scale = jnp.float32(1.0 / _K)
    out_ref[0] = jnp.concatenate([acc0, acc1, acc2], axis=1) * scale


def _vnlr(x, d_t, n):
    d = jnp.dot(x, d_t, preferred_element_type=jnp.float32)
    z = x * d
    dot = z[0:n] + z[n:2 * n] + z[2 * n:3 * n]
    zz = d * d
    d2 = zz[0:n] + zz[n:2 * n] + zz[2 * n:3 * n]
    coef = dot / (d2 + _EPS)
    dot3 = jnp.concatenate([dot, dot, dot], axis=0)
    coef3 = jnp.concatenate([coef, coef, coef], axis=0)
    return jnp.where(dot3 >= 0, x, x - coef3 * d)


def _resblock(x, a0t, f0t, a1t, f1t, sct, n):
    net = jnp.dot(_vnlr(x, a0t, n), f0t, preferred_element_type=jnp.float32)
    dx = jnp.dot(_vnlr(net, a1t, n), f1t, preferred_element_type=jnp.float32)
    return jnp.dot(x, sct, preferred_element_type=jnp.float32) + dx


def _poolcat(x, n):
    m0 = jnp.mean(x[0:n], axis=0, keepdims=True)
    m1 = jnp.mean(x[n:2 * n], axis=0, keepdims=True)
    m2 = jnp.mean(x[2 * n:3 * n], axis=0, keepdims=True)
    c = x.shape[1]
    pooled = jnp.concatenate([
        jnp.broadcast_to(m0, (n, c)),
        jnp.broadcast_to(m1, (n, c)),
        jnp.broadcast_to(m2, (n, c)),
    ], axis=0)
    return jnp.concatenate([x, pooled], axis=1)


def _dense_kernel(net0_ref, fcpos_t_ref, *rest):
    wrefs = rest[:-3]
    actc_t_ref, fcc_t_ref, out_ref = rest[-3:]
    n = _N
    x = net0_ref[0]
    x = jnp.concatenate([x[:, 0:64], x[:, 64:128], x[:, 128:192]],
                        axis=0)
    net = jnp.dot(x, fcpos_t_ref[...], preferred_element_type=jnp.float32)
    for i in range(5):
        a0t = wrefs[5 * i][...]
        f0t = wrefs[5 * i + 1][...]
        a1t = wrefs[5 * i + 2][...]
        f1t = wrefs[5 * i + 3][...]
        sct = wrefs[5 * i + 4][...]
        net = _resblock(net, a0t, f0t, a1t, f1t, sct, n)
        if i < 4:
            net = _poolcat(net, n)
    m0 = jnp.mean(net[0:n], axis=0, keepdims=True)
    m1 = jnp.mean(net[n:2 * n], axis=0, keepdims=True)
    m2 = jnp.mean(net[2 * n:3 * n], axis=0, keepdims=True)
    m3 = jnp.concatenate([m0, m1, m2], axis=0)
    d = jnp.dot(m3, actc_t_ref[...], preferred_element_type=jnp.float32)
    z = m3 * d
    dot = jnp.sum(z, axis=0, keepdims=True)
    d2 = jnp.sum(d * d, axis=0, keepdims=True)
    out = jnp.where(dot >= 0, m3, m3 - (dot / (d2 + _EPS)) * d)
    out_ref[0] = jnp.dot(out, fcc_t_ref[...],
                         preferred_element_type=jnp.float32)


def kernel(inputs, params):
    pts = inputs

    wf = params['conv_pos_feat']
    wd = params['conv_pos_dir']
    w2 = jnp.stack([wf.T, wd.T], axis=1)
    eye3 = jnp.eye(3, dtype=jnp.float32)
    mw = jnp.einsum('cso,vw->cvwso', w2, eye3).reshape(9, 384)

    nt = _N // _TQ
    net0 = pl.pallas_call(
        _knn_edge_kernel,
        grid=(_B, nt),
        in_specs=[
            pl.BlockSpec((1, _N, 3), lambda b, t: (b, 0, 0)),
            pl.BlockSpec((1, _TQ, 3), lambda b, t: (b, t, 0)),
            pl.BlockSpec((9, 384), lambda b, t: (0, 0)),
        ],
        out_specs=pl.BlockSpec((1, _TQ, 192), lambda b, t: (b, t, 0)),
        out_shape=jax.ShapeDtypeStruct((_B, _N, 192), jnp.float32),
    )(pts, pts, mw)

    wlist = [params['fc_pos'].T]
    for blk in params['blocks']:
        wlist += [blk['act0'].T, blk['fc0'].T, blk['act1'].T,
                  blk['fc1'].T, blk['sc'].T]
    wlist += [params['actc_dir'].T, params['fc_c'].T]

    wspecs = [pl.BlockSpec(w.shape, lambda b: (0,) * w.ndim) for w in wlist]
    cvec = pl.pallas_call(
        _dense_kernel,
        grid=(_B,),
        in_specs=[pl.BlockSpec((1, _N, 192), lambda b: (b, 0, 0))] + wspecs,
        out_specs=pl.BlockSpec((1, 3, 128), lambda b: (b, 0, 0)),
        out_shape=jax.ShapeDtypeStruct((_B, 3, 128), jnp.float32),
    )(net0, *wlist)

    return jnp.transpose(cvec, (0, 2, 1)).reshape(_B, 384)


# speedup vs baseline: 7.0138x; 1.0843x over previous
import jax
import jax.numpy as jnp
from jax.experimental import pallas as pl
from jax.experimental.pallas import tpu as pltpu

_EPS = 1e-6
_K = 20
_B, _N = 8, 2048
_TQ = 1024


def _knn_edge_kernel(pts_ref, xq_ref, m_ref, out_ref):
    pts = pts_ref[0]
    xq = xq_ref[0]
    mw = m_ref[...]

    dotq = jax.lax.dot_general(xq, pts, (((1,), (1,)), ((), ())),
                               preferred_element_type=jnp.float32)
    inner = -2.0 * dotq
    xxq = jnp.sum(xq * xq, axis=1, keepdims=True)
    sq = pts * pts
    ones13 = jnp.ones((1, 3), jnp.float32)
    xxall = jax.lax.dot_general(ones13, sq, (((1,), (1,)), ((), ())),
                                preferred_element_type=jnp.float32,
                                precision=jax.lax.Precision.HIGHEST)
    pd = -xxq - inner - xxall

    pts_hi = pts.astype(jnp.bfloat16).astype(jnp.float32)
    pts_hilo = jnp.concatenate([pts_hi, pts - pts_hi], axis=1)

    iota = jax.lax.broadcasted_iota(jnp.int32, pd.shape, 1)
    big = jnp.int32(_N)
    neg_inf = jnp.float32(-jnp.inf)

    xqx = xq[:, 0:1]
    xqy = xq[:, 1:2]
    xqz = xq[:, 2:3]

    work = pd
    acc0 = jnp.zeros((_TQ, 64), jnp.float32)
    acc1 = jnp.zeros((_TQ, 64), jnp.float32)
    acc2 = jnp.zeros((_TQ, 64), jnp.float32)
    for _ in range(_K):
        mx = jnp.max(work, axis=1, keepdims=True)
        cand = jnp.where(work == mx, iota, big)
        am = jnp.min(cand, axis=1, keepdims=True)
        hit = cand == am
        work = jnp.where(hit, neg_inf, work)
        onehot = hit.astype(jnp.float32)
        f6 = jnp.dot(onehot, pts_hilo,
                     preferred_element_type=jnp.float32)
        f = f6[:, 0:3] + f6[:, 3:6]
        fx = f[:, 0:1]
        fy = f[:, 1:2]
        fz = f[:, 2:3]
        dx = fx - xqx
        dy = fy - xqy
        dz = fz - xqz
        cx = fy * xqz - fz * xqy
        cy = fz * xqx - fx * xqz
        cz = fx * xqy - fy * xqx
        g = jnp.concatenate([dx, dy, dz, xqx, xqy, xqz, cx, cy, cz],
                            axis=1)
        pdm = jnp.dot(g, mw, preferred_element_type=jnp.float32)
        p0 = pdm[:, 0:64]
        d0 = pdm[:, 64:128]
        p1 = pdm[:, 128:192]
        d1 = pdm[:, 192:256]
        p2 = pdm[:, 256:320]
        d2v = pdm[:, 320:384]
        dot = p0 * d0 + p1 * d1 + p2 * d2v
        dd = d0 * d0 + d1 * d1 + d2v * d2v
        coef = dot / (dd + _EPS)
        keep = dot >= 0
        acc0 = acc0 + jnp.where(keep, p0, p0 - coef * d0)
        acc1 = acc1 + jnp.where(keep, p1, p1 - coef * d1)
        acc2 = acc2 + jnp.where(keep, p2, p2 - coef * d2v)

    scale = jnp.float32(1.0 / _K)
    out_ref[0] = jnp.concatenate([acc0, acc1, acc2], axis=1) * scale


def _vnlr(x, d_t, n):
    d = jnp.dot(x, d_t, preferred_element_type=jnp.float32)
    z = x * d
    dot = z[0:n] + z[n:2 * n] + z[2 * n:3 * n]
    zz = d * d
    d2 = zz[0:n] + zz[n:2 * n] + zz[2 * n:3 * n]
    coef = dot / (d2 + _EPS)
    dot3 = jnp.concatenate([dot, dot, dot], axis=0)
    coef3 = jnp.concatenate([coef, coef, coef], axis=0)
    return jnp.where(dot3 >= 0, x, x - coef3 * d)


def _resblock(x, a0t, f0t, a1t, f1t, sct, n):
    net = jnp.dot(_vnlr(x, a0t, n), f0t, preferred_element_type=jnp.float32)
    dx = jnp.dot(_vnlr(net, a1t, n), f1t, preferred_element_type=jnp.float32)
    return jnp.dot(x, sct, preferred_element_type=jnp.float32) + dx


def _poolcat(x, n):
    m0 = jnp.mean(x[0:n], axis=0, keepdims=True)
    m1 = jnp.mean(x[n:2 * n], axis=0, keepdims=True)
    m2 = jnp.mean(x[2 * n:3 * n], axis=0, keepdims=True)
    c = x.shape[1]
    pooled = jnp.concatenate([
        jnp.broadcast_to(m0, (n, c)),
        jnp.broadcast_to(m1, (n, c)),
        jnp.broadcast_to(m2, (n, c)),
    ], axis=0)
    return jnp.concatenate([x, pooled], axis=1)


def _dense_kernel(net0_ref, fcpos_t_ref, *rest):
    wrefs = rest[:-3]
    actc_t_ref, fcc_t_ref, out_ref = rest[-3:]
    n = _N
    x = net0_ref[0]
    x = jnp.concatenate([x[:, 0:64], x[:, 64:128], x[:, 128:192]],
                        axis=0)
    net = jnp.dot(x, fcpos_t_ref[...], preferred_element_type=jnp.float32)
    for i in range(5):
        a0t = wrefs[5 * i][...]
        f0t = wrefs[5 * i + 1][...]
        a1t = wrefs[5 * i + 2][...]
        f1t = wrefs[5 * i + 3][...]
        sct = wrefs[5 * i + 4][...]
        net = _resblock(net, a0t, f0t, a1t, f1t, sct, n)
        if i < 4:
            net = _poolcat(net, n)
    m0 = jnp.mean(net[0:n], axis=0, keepdims=True)
    m1 = jnp.mean(net[n:2 * n], axis=0, keepdims=True)
    m2 = jnp.mean(net[2 * n:3 * n], axis=0, keepdims=True)
    m3 = jnp.concatenate([m0, m1, m2], axis=0)
    d = jnp.dot(m3, actc_t_ref[...], preferred_element_type=jnp.float32)
    z = m3 * d
    dot = jnp.sum(z, axis=0, keepdims=True)
    d2 = jnp.sum(d * d, axis=0, keepdims=True)
    out = jnp.where(dot >= 0, m3, m3 - (dot / (d2 + _EPS)) * d)
    out_ref[0] = jnp.dot(out, fcc_t_ref[...],
                         preferred_element_type=jnp.float32)


def kernel(inputs, params):
    pts = inputs

    wf = params['conv_pos_feat']
    wd = params['conv_pos_dir']
    w2 = jnp.stack([wf.T, wd.T], axis=1)
    eye3 = jnp.eye(3, dtype=jnp.float32)
    mw = jnp.einsum('cso,vw->cvwso', w2, eye3).reshape(9, 384)

    nt = _N // _TQ
    net0 = pl.pallas_call(
        _knn_edge_kernel,
        grid=(_B, nt),
        in_specs=[
            pl.BlockSpec((1, _N, 3), lambda b, t: (b, 0, 0)),
            pl.BlockSpec((1, _TQ, 3), lambda b, t: (b, t, 0)),
            pl.BlockSpec((9, 384), lambda b, t: (0, 0)),
        ],
        out_specs=pl.BlockSpec((1, _TQ, 192), lambda b, t: (b, t, 0)),
        out_shape=jax.ShapeDtypeStruct((_B, _N, 192), jnp.float32),
    )(pts, pts, mw)

    wlist = [params['fc_pos'].T]
    for blk in params['blocks']:
        wlist += [blk['act0'].T, blk['fc0'].T, blk['act1'].T,
                  blk['fc1'].T, blk['sc'].T]
    wlist += [params['actc_dir'].T, params['fc_c'].T]

    wspecs = [pl.BlockSpec(w.shape, lambda b: (0,) * w.ndim) for w in wlist]
    cvec = pl.pallas_call(
        _dense_kernel,
        grid=(_B,),
        in_specs=[pl.BlockSpec((1, _N, 192), lambda b: (b, 0, 0))] + wspecs,
        out_specs=pl.BlockSpec((1, 3, 128), lambda b: (b, 0, 0)),
        out_shape=jax.ShapeDtypeStruct((_B, 3, 128), jnp.float32),
    )(net0, *wlist)

    return jnp.transpose(cvec, (0, 2, 1)).reshape(_B, 384)
